# baseline (speedup 1.0000x reference)
import functools
import math

import jax
import jax.numpy as jnp
from jax import lax
from jax.experimental import pallas as pl
from jax.experimental.pallas import tpu as pltpu
from jax.experimental.pallas import tpu_sc as plsc

N = 10000
M = 320000
D = 128
H = 8
DH = 16
HD2 = D // 2

EPT = M // 16
C = 80
NCHUNK = EPT // C
NGRP = C // 16
NP = 10240
NP2 = NP // 2
NP16 = NP // 16
NROWS_T = NP2 // 16
DROWS_T = NP16 // 16


def _proj_body(x_ref, wq_ref, wk_ref, wv_ref, bq_ref, bk_ref, bv_ref,
               qt_ref, kvt_ref):
    x = x_ref[...]
    dn = (((1,), (1,)), ((), ()))
    q = lax.dot_general(x, wq_ref[...], dn, preferred_element_type=jnp.float32)
    q = (q + bq_ref[...]) * (1.0 / math.sqrt(DH))
    k = lax.dot_general(x, wk_ref[...], dn, preferred_element_type=jnp.float32)
    k = k + bk_ref[...]
    v = lax.dot_general(x, wv_ref[...], dn, preferred_element_type=jnp.float32)
    v = v + bv_ref[...]
    zpad = jnp.zeros((N, HD2), jnp.float32)
    qt_ref[0] = lax.concatenate([q[:, :HD2], zpad], 1)
    qt_ref[1] = lax.concatenate([q[:, HD2:], zpad], 1)
    kvt_ref[0] = lax.concatenate([k[:, :HD2], v[:, :HD2]], 1)
    kvt_ref[1] = lax.concatenate([k[:, HD2:], v[:, HD2:]], 1)


def _final_body(num_ref, den_ref, wo_ref, bo_ref, o_ref):
    num = num_ref[:N, :]
    den = (den_ref[0] + den_ref[1])[:N, :]
    lane = lax.broadcasted_iota(jnp.int32, (H, D), 1)
    row = lax.broadcasted_iota(jnp.int32, (H, D), 0)
    expand = jnp.where(lane // DH == row, 1.0, 0.0).astype(jnp.float32)
    den_b = lax.dot_general(den, expand, (((1,), (0,)), ((), ())),
                            preferred_element_type=jnp.float32)
    attn_out = num / (den_b + 1e-12)
    dn = (((1,), (1,)), ((), ()))
    o_ref[...] = lax.dot_general(attn_out, wo_ref[...], dn,
                                 preferred_element_type=jnp.float32) + bo_ref[...]


def _lane_gather(x, idx):
    dnums = lax.GatherDimensionNumbers(
        offset_dims=(), collapsed_slice_dims=(0,), start_index_map=(0,))
    return lax.gather(x, idx[:, None], dnums, (1,),
                      mode=lax.GatherScatterMode.PROMISE_IN_BOUNDS)


def _edge_body(qt_hbm, kvt_hbm, ed_hbm, aw_hbm, zeros_hbm,
               num_hbm, den_hbm,
               ed_v, aw_v, qr_v, kvr_v, on_v, od_v, acc_num, acc_den,
               sem_e, sem_g, sem_s):
    cid = lax.axis_index("c")
    sid = lax.axis_index("s")
    row0 = sid * NROWS_T
    drow0 = sid * DROWS_T

    pltpu.sync_copy(zeros_hbm.at[pl.ds(row0, NROWS_T)],
                    acc_num.at[pl.ds(row0, NROWS_T)])
    pltpu.sync_copy(zeros_hbm.at[pl.ds(drow0, DROWS_T)],
                    acc_den.at[pl.ds(drow0, DROWS_T)])

    base0 = sid * EPT
    coff = cid * N
    lane = lax.iota(jnp.int32, 16)
    zv = jnp.zeros((16,), jnp.float32)

    def ed_issue(t, s):
        ch = sid * NCHUNK + jnp.minimum(t, NCHUNK - 1)
        pltpu.async_copy(ed_hbm.at[cid, ch], ed_v[s], sem_e[s])
        pltpu.async_copy(aw_hbm.at[cid, ch], aw_v[s], sem_e[s])

    def ed_wait(s):
        pltpu.make_async_copy(ed_hbm.at[0, 0], ed_v[s], sem_e[s]).wait()
        pltpu.make_async_copy(aw_hbm.at[0, 0], aw_v[s], sem_e[s]).wait()

    def g_issue(s):
        pltpu.async_copy(qt_hbm.at[ed_v[s].at[0]], qr_v[s], sem_g[s])
        pltpu.async_copy(kvt_hbm.at[ed_v[s].at[1]], kvr_v[s], sem_g[s])

    def g_wait(s):
        pltpu.make_async_copy(qt_hbm.at[ed_v[s].at[0]], qr_v[s],
                              sem_g[s]).wait()
        pltpu.make_async_copy(kvt_hbm.at[ed_v[s].at[1]], kvr_v[s],
                              sem_g[s]).wait()

    def scat_issue(s):
        pltpu.async_copy(on_v, acc_num.at[ed_v[s].at[2]], sem_s, add=True)
        pltpu.async_copy(od_v, acc_den.at[ed_v[s].at[3]], sem_s, add=True)

    def scat_wait(s):
        pltpu.make_async_copy(on_v, acc_num.at[ed_v[s].at[2]], sem_s).wait()
        pltpu.make_async_copy(od_v, acc_den.at[ed_v[s].at[3]], sem_s).wait()

    def comp(s):
        def group(g, carry):
            e0 = g * 16
            sl = pl.ds(e0, 16)
            qiv = ed_v[s][0, sl] - coff
            awf = [aw_v[s][hl, sl] for hl in range(4)]
            for j in range(16):
                e = e0 + j
                qis = qiv[j]
                par = qis & 1
                slot = qis & 15
                den_e = zv
                den_o = zv
                for hl in range(4):
                    qh = qr_v[s][e, pl.ds(hl * DH, DH)]
                    kh = kvr_v[s][e, pl.ds(hl * DH, DH)]
                    vh = kvr_v[s][e, pl.ds(HD2 + hl * DH, DH)]
                    r = qh * kh
                    for step in (8, 4, 2, 1):
                        r = r + _lane_gather(r, lane ^ step)
                    p = jnp.exp(r + awf[hl][j])
                    pv = p * vh
                    for ps in (0, 1):
                        mp = jnp.where(par == ps, 1.0, 0.0)
                        on_v[e, pl.ds(ps * HD2 + hl * DH, DH)] = pv * mp
                    den_e = den_e + jnp.where(lane == cid * 4 + hl, p, 0.0)
                    den_o = den_o + jnp.where(lane == 8 + cid * 4 + hl, p, 0.0)
                for s8 in range(8):
                    me = jnp.where(slot == 2 * s8, 1.0, 0.0)
                    mo = jnp.where(slot == 2 * s8 + 1, 1.0, 0.0)
                    od_v[e, pl.ds(s8 * DH, DH)] = den_e * me + den_o * mo
            return carry
        lax.fori_loop(0, NGRP, group, 0)

    def phase(t, s, ns):
        g_wait(s)
        comp(s)
        scat_issue(s)
        ed_wait(ns)
        g_issue(ns)
        scat_wait(s)
        ed_issue(t + 2, s)

    plsc.subcore_barrier()
    ed_issue(0, 0)
    ed_issue(1, 1)
    ed_wait(0)
    g_issue(0)

    def pair(ib, carry):
        phase(2 * ib, 0, 1)
        phase(2 * ib + 1, 1, 0)
        return carry

    lax.fori_loop(0, NCHUNK // 2, pair, 0)

    g_wait(0)
    ed_wait(1)

    plsc.subcore_barrier()
    pltpu.sync_copy(acc_num.at[pl.ds(row0, NROWS_T)],
                    num_hbm.at[cid, pl.ds(row0, NROWS_T)])
    pltpu.sync_copy(acc_den.at[pl.ds(drow0, DROWS_T)],
                    den_hbm.at[cid, pl.ds(drow0, DROWS_T)])


def _pair(ty):
    return (ty, ty)


_edge_kernel = functools.partial(
    pl.kernel,
    out_type=(jax.ShapeDtypeStruct((2, NP2, D), jnp.float32),
              jax.ShapeDtypeStruct((2, NP16, D), jnp.float32)),
    mesh=plsc.VectorSubcoreMesh(core_axis_name="c", subcore_axis_name="s"),
    scratch_types=[
        _pair(pltpu.VMEM((4, C), jnp.int32)),
        _pair(pltpu.VMEM((4, C), jnp.float32)),
        _pair(pltpu.VMEM((C, D), jnp.float32)),
        _pair(pltpu.VMEM((C, D), jnp.float32)),
        pltpu.VMEM((C, D), jnp.float32),
        pltpu.VMEM((C, D), jnp.float32),
        pltpu.VMEM_SHARED((NP2, D), jnp.float32),
        pltpu.VMEM_SHARED((NP16, D), jnp.float32),
        _pair(pltpu.SemaphoreType.DMA),
        _pair(pltpu.SemaphoreType.DMA),
        pltpu.SemaphoreType.DMA,
    ],
)(_edge_body)


@jax.jit
def kernel(query, edges, attn_weights, w_q, w_k, w_v, b_q, b_k, b_v,
           w_out, b_out):
    qt, kvt = pl.pallas_call(
        _proj_body,
        out_shape=(jax.ShapeDtypeStruct((2, N, D), jnp.float32),
                   jax.ShapeDtypeStruct((2, N, D), jnp.float32)),
    )(query, w_q, w_k, w_v,
      b_q.reshape(1, D), b_k.reshape(1, D), b_v.reshape(1, D))
    qt = qt.reshape(2 * N, D)
    kvt = kvt.reshape(2 * N, D)

    edges = edges.astype(jnp.int32)
    qi = edges[0]
    kj = edges[1]
    r1 = jnp.right_shift(qi, 1)
    r3 = jnp.right_shift(qi, 4)
    packs = []
    aws = []
    for c in (0, 1):
        cols = jnp.stack([qi + c * N, kj + c * N, r1, r3], axis=1)
        packs.append(cols.reshape(M // C, C, 4).transpose(0, 2, 1))
        aws.append(attn_weights[:, 4 * c:4 * c + 4]
                   .reshape(M // C, C, 4).transpose(0, 2, 1))
    ed = jnp.stack(packs)
    aw4 = jnp.stack(aws)
    zeros = jnp.zeros((NP2, D), jnp.float32)
    nd_num, nd_den = _edge_kernel(qt, kvt, ed, aw4, zeros)

    numr = nd_num.reshape(2, NP, HD2)
    num_full = jnp.concatenate([numr[0], numr[1]], axis=1)
    den_r = nd_den.reshape(2, NP, H)

    out = pl.pallas_call(
        _final_body,
        out_shape=jax.ShapeDtypeStruct((N, D), jnp.float32),
    )(num_full, den_r, w_out, b_out.reshape(1, D))
    return out


# speedup vs baseline: 1.0889x; 1.0889x over previous
import functools
import math

import jax
import jax.numpy as jnp
from jax import lax
from jax.experimental import pallas as pl
from jax.experimental.pallas import tpu as pltpu
from jax.experimental.pallas import tpu_sc as plsc

N = 10000
M = 320000
D = 128
H = 8
DH = 16
HD2 = D // 2

EPT = M // 16
C = 80
NCHUNK = EPT // C
NGRP = C // 16
NP = 10240
NP2 = NP // 2
NP16 = NP // 16
NROWS_T = NP2 // 16
DROWS_T = NP16 // 16


def _proj_body(x_ref, wq_ref, wk_ref, wv_ref, bq_ref, bk_ref, bv_ref,
               qt_ref, kvt_ref):
    x = x_ref[...]
    dn = (((1,), (1,)), ((), ()))
    q = lax.dot_general(x, wq_ref[...], dn, preferred_element_type=jnp.float32)
    q = (q + bq_ref[...]) * (1.0 / math.sqrt(DH))
    k = lax.dot_general(x, wk_ref[...], dn, preferred_element_type=jnp.float32)
    k = k + bk_ref[...]
    v = lax.dot_general(x, wv_ref[...], dn, preferred_element_type=jnp.float32)
    v = v + bv_ref[...]
    zpad = jnp.zeros((N, HD2), jnp.float32)
    qt_ref[0] = lax.concatenate([q[:, :HD2], zpad], 1)
    qt_ref[1] = lax.concatenate([q[:, HD2:], zpad], 1)
    kvt_ref[0] = lax.concatenate([k[:, :HD2], v[:, :HD2]], 1)
    kvt_ref[1] = lax.concatenate([k[:, HD2:], v[:, HD2:]], 1)


def _final_body(num_ref, den_ref, wo_ref, bo_ref, o_ref):
    num = num_ref[:N, :]
    den = (den_ref[0] + den_ref[1])[:N, :]
    lane = lax.broadcasted_iota(jnp.int32, (H, D), 1)
    row = lax.broadcasted_iota(jnp.int32, (H, D), 0)
    expand = jnp.where(lane // DH == row, 1.0, 0.0).astype(jnp.float32)
    den_b = lax.dot_general(den, expand, (((1,), (0,)), ((), ())),
                            preferred_element_type=jnp.float32)
    attn_out = num / (den_b + 1e-12)
    dn = (((1,), (1,)), ((), ()))
    o_ref[...] = lax.dot_general(attn_out, wo_ref[...], dn,
                                 preferred_element_type=jnp.float32) + bo_ref[...]


def _lane_gather(x, idx):
    dnums = lax.GatherDimensionNumbers(
        offset_dims=(), collapsed_slice_dims=(0,), start_index_map=(0,))
    return lax.gather(x, idx[:, None], dnums, (1,),
                      mode=lax.GatherScatterMode.PROMISE_IN_BOUNDS)


def _edge_body(qt_hbm, kvt_hbm, ed_hbm, aw_hbm, zeros_hbm,
               num_hbm, den_hbm,
               ed_v, aw_v, r1s_v, r3s_v, qr_v, kvr_v, on_v, od_v,
               acc_num, acc_den,
               sem_e, sem_g, sem_s):
    cid = lax.axis_index("c")
    sid = lax.axis_index("s")
    row0 = sid * NROWS_T
    drow0 = sid * DROWS_T

    pltpu.sync_copy(zeros_hbm.at[pl.ds(row0, NROWS_T)],
                    acc_num.at[pl.ds(row0, NROWS_T)])
    pltpu.sync_copy(zeros_hbm.at[pl.ds(drow0, DROWS_T)],
                    acc_den.at[pl.ds(drow0, DROWS_T)])

    base0 = sid * EPT
    coff = cid * N
    lane = lax.iota(jnp.int32, 16)
    zv = jnp.zeros((16,), jnp.float32)
    zvi = jnp.zeros((16,), jnp.int32)

    def ed_issue(t, s):
        ch = sid * NCHUNK + jnp.minimum(t, NCHUNK - 1)
        pltpu.async_copy(ed_hbm.at[cid, ch], ed_v[s], sem_e[s])
        pltpu.async_copy(aw_hbm.at[cid, ch], aw_v[s], sem_e[s])

    def ed_wait(s):
        pltpu.make_async_copy(ed_hbm.at[0, 0], ed_v[s], sem_e[s]).wait()
        pltpu.make_async_copy(aw_hbm.at[0, 0], aw_v[s], sem_e[s]).wait()

    def g_issue(s):
        pltpu.async_copy(qt_hbm.at[ed_v[s].at[0]], qr_v[s], sem_g[s])
        pltpu.async_copy(kvt_hbm.at[ed_v[s].at[1]], kvr_v[s], sem_g[s])

    def g_wait(s):
        pltpu.make_async_copy(qt_hbm.at[ed_v[s].at[0]], qr_v[s],
                              sem_g[s]).wait()
        pltpu.make_async_copy(kvt_hbm.at[ed_v[s].at[1]], kvr_v[s],
                              sem_g[s]).wait()

    def scat_issue():
        pltpu.async_copy(on_v, acc_num.at[r1s_v], sem_s, add=True)
        pltpu.async_copy(od_v, acc_den.at[r3s_v], sem_s, add=True)

    def scat_wait():
        pltpu.make_async_copy(on_v, acc_num.at[r1s_v], sem_s).wait()
        pltpu.make_async_copy(od_v, acc_den.at[r3s_v], sem_s).wait()

    def comp(s):
        def group(g, carry):
            e0 = g * 16
            sl = pl.ds(e0, 16)
            qiv = ed_v[s][0, sl] - coff
            r1s_v[sl] = ed_v[s][2, sl]
            r3s_v[sl] = ed_v[s][3, sl]
            awf = [aw_v[s][hl, sl] for hl in range(4)]
            for j in range(16):
                e = e0 + j
                qis = qiv[j]
                par = qis & 1
                slot = qis & 15
                den_e = zv
                den_o = zv
                for hl in range(4):
                    qh = qr_v[s][e, pl.ds(hl * DH, DH)]
                    kh = kvr_v[s][e, pl.ds(hl * DH, DH)]
                    vh = kvr_v[s][e, pl.ds(HD2 + hl * DH, DH)]
                    r = qh * kh
                    for step in (8, 4, 2, 1):
                        r = r + _lane_gather(r, lane ^ step)
                    p = jnp.exp(r + awf[hl][j])
                    pv = p * vh
                    for ps in (0, 1):
                        mp = jnp.where(par == ps, 1.0, 0.0)
                        on_v[e, pl.ds(ps * HD2 + hl * DH, DH)] = pv * mp
                    den_e = den_e + jnp.where(lane == cid * 4 + hl, p, 0.0)
                    den_o = den_o + jnp.where(lane == 8 + cid * 4 + hl, p, 0.0)
                for s8 in range(8):
                    me = jnp.where(slot == 2 * s8, 1.0, 0.0)
                    mo = jnp.where(slot == 2 * s8 + 1, 1.0, 0.0)
                    od_v[e, pl.ds(s8 * DH, DH)] = den_e * me + den_o * mo
            return carry
        lax.fori_loop(0, NGRP, group, 0)

    def phase(t, s, ns):
        g_wait(s)
        ed_wait(ns)
        g_issue(ns)
        scat_wait()
        comp(s)
        scat_issue()
        ed_issue(t + 2, s)

    plsc.subcore_barrier()
    ed_issue(0, 0)
    ed_issue(1, 1)
    ed_wait(0)
    g_issue(0)
    pltpu.sync_copy(zeros_hbm.at[pl.ds(0, C)], on_v)
    pltpu.sync_copy(zeros_hbm.at[pl.ds(0, C)], od_v)

    def zgrp(g, carry):
        sl = pl.ds(g * 16, 16)
        r1s_v[sl] = zvi
        r3s_v[sl] = zvi
        return carry
    lax.fori_loop(0, NGRP, zgrp, 0)
    scat_issue()

    def pair(ib, carry):
        phase(2 * ib, 0, 1)
        phase(2 * ib + 1, 1, 0)
        return carry

    lax.fori_loop(0, NCHUNK // 2, pair, 0)

    scat_wait()
    g_wait(0)
    ed_wait(1)

    plsc.subcore_barrier()
    pltpu.sync_copy(acc_num.at[pl.ds(row0, NROWS_T)],
                    num_hbm.at[cid, pl.ds(row0, NROWS_T)])
    pltpu.sync_copy(acc_den.at[pl.ds(drow0, DROWS_T)],
                    den_hbm.at[cid, pl.ds(drow0, DROWS_T)])


def _pair(ty):
    return (ty, ty)


_edge_kernel = functools.partial(
    pl.kernel,
    out_type=(jax.ShapeDtypeStruct((2, NP2, D), jnp.float32),
              jax.ShapeDtypeStruct((2, NP16, D), jnp.float32)),
    mesh=plsc.VectorSubcoreMesh(core_axis_name="c", subcore_axis_name="s"),
    scratch_types=[
        _pair(pltpu.VMEM((4, C), jnp.int32)),
        _pair(pltpu.VMEM((4, C), jnp.float32)),
        pltpu.VMEM((C,), jnp.int32),
        pltpu.VMEM((C,), jnp.int32),
        _pair(pltpu.VMEM((C, D), jnp.float32)),
        _pair(pltpu.VMEM((C, D), jnp.float32)),
        pltpu.VMEM((C, D), jnp.float32),
        pltpu.VMEM((C, D), jnp.float32),
        pltpu.VMEM_SHARED((NP2, D), jnp.float32),
        pltpu.VMEM_SHARED((NP16, D), jnp.float32),
        _pair(pltpu.SemaphoreType.DMA),
        _pair(pltpu.SemaphoreType.DMA),
        pltpu.SemaphoreType.DMA,
    ],
)(_edge_body)


@jax.jit
def kernel(query, edges, attn_weights, w_q, w_k, w_v, b_q, b_k, b_v,
           w_out, b_out):
    qt, kvt = pl.pallas_call(
        _proj_body,
        out_shape=(jax.ShapeDtypeStruct((2, N, D), jnp.float32),
                   jax.ShapeDtypeStruct((2, N, D), jnp.float32)),
    )(query, w_q, w_k, w_v,
      b_q.reshape(1, D), b_k.reshape(1, D), b_v.reshape(1, D))
    qt = qt.reshape(2 * N, D)
    kvt = kvt.reshape(2 * N, D)

    edges = edges.astype(jnp.int32)
    qi = edges[0]
    kj = edges[1]
    r1 = jnp.right_shift(qi, 1)
    r3 = jnp.right_shift(qi, 4)
    packs = []
    aws = []
    for c in (0, 1):
        cols = jnp.stack([qi + c * N, kj + c * N, r1, r3], axis=1)
        packs.append(cols.reshape(M // C, C, 4).transpose(0, 2, 1))
        aws.append(attn_weights[:, 4 * c:4 * c + 4]
                   .reshape(M // C, C, 4).transpose(0, 2, 1))
    ed = jnp.stack(packs)
    aw4 = jnp.stack(aws)
    zeros = jnp.zeros((NP2, D), jnp.float32)
    nd_num, nd_den = _edge_kernel(qt, kvt, ed, aw4, zeros)

    numr = nd_num.reshape(2, NP, HD2)
    num_full = jnp.concatenate([numr[0], numr[1]], axis=1)
    den_r = nd_den.reshape(2, NP, H)

    out = pl.pallas_call(
        _final_body,
        out_shape=jax.ShapeDtypeStruct((N, D), jnp.float32),
    )(num_full, den_r, w_out, b_out.reshape(1, D))
    return out
